# Initial kernel scaffold; baseline (speedup 1.0000x reference)
import jax
import jax.numpy as jnp
from jax.experimental import pallas as pl


def kernel(x, Wqkv, Wout, slot_k_init, slot_v_init, Wg, bg, Wf, bf, log_tau_read, log_tau_write):
    raise NotImplementedError("write your pallas kernel here")


# speedup vs baseline: 9.4548x; 9.4548x over previous
import jax
import jax.numpy as jnp
import numpy as np
from jax.experimental import pallas as pl
from jax.experimental.pallas import tpu as pltpu

B, T, D = 2, 2048, 1024
H, DH, K = 16, 64, 128
RT = 8
TQ = 256
NEG = -1e30


def _attn_kernel(ltau_ref, x_ref, wq_ref, wk_ref, wv_ref, sk_ref, sv_ref,
                 wf_ref, bf_ref, wo_ref, o_ref, q_s, k_s, v_s, yh_s):
    h = pl.program_id(1)
    xb = x_ref[0]

    q_s[...] = jax.lax.dot_general(xb, wq_ref[...], (((1,), (1,)), ((), ())),
                                   preferred_element_type=jnp.float32)
    k_s[...] = jax.lax.dot_general(xb, wk_ref[...], (((1,), (1,)), ((), ())),
                                   preferred_element_type=jnp.float32)
    v_s[...] = jax.lax.dot_general(xb, wv_ref[...], (((1,), (1,)), ((), ())),
                                   preferred_element_type=jnp.float32)
    q = q_s[...]

    tau = jnp.exp(ltau_ref[0, 0])
    qn = q / jnp.maximum(jnp.sqrt(jnp.sum(q * q, axis=-1, keepdims=True)), 1e-12)
    sk = sk_ref[0]
    sk = sk / jnp.maximum(jnp.sqrt(jnp.sum(sk * sk, axis=-1, keepdims=True)), 1e-12)
    logits = jax.lax.dot_general(qn, sk, (((1,), (1,)), ((), ())),
                                 preferred_element_type=jnp.float32) * tau
    col = jax.lax.broadcasted_iota(jnp.int32, (T, K), 1)
    work = logits
    selmask = jnp.zeros((T, K), jnp.bool_)
    for _ in range(RT):
        m = jnp.max(work, axis=-1, keepdims=True)
        ism = work >= m
        first = jnp.min(jnp.where(ism, col, K), axis=-1, keepdims=True)
        sel = col == first
        selmask = jnp.logical_or(selmask, sel)
        work = jnp.where(sel, NEG, work)
    lm = jnp.where(selmask, logits, NEG)
    mx = jnp.max(lm, axis=-1, keepdims=True)
    e = jnp.where(selmask, jnp.exp(lm - mx), 0.0)
    p = e / jnp.sum(e, axis=-1, keepdims=True)
    read = jax.lax.dot_general(p, sv_ref[0], (((1,), (0,)), ((), ())),
                               preferred_element_type=jnp.float32)

    scale = 1.0 / np.sqrt(DH)
    for qt in range(T // TQ):
        qtile = q_s[pl.ds(qt * TQ, TQ), :]
        rowp = qt * TQ + jax.lax.broadcasted_iota(jnp.int32, (TQ, TQ), 0)

        def body(kt, carry, qtile=qtile, rowp=rowp):
            acc, m0, l0 = carry
            kblk = k_s[pl.ds(kt * TQ, TQ), :]
            s = jax.lax.dot_general(qtile, kblk, (((1,), (1,)), ((), ())),
                                    preferred_element_type=jnp.float32) * scale
            colp = kt * TQ + jax.lax.broadcasted_iota(jnp.int32, (TQ, TQ), 1)
            s = jnp.where(colp > rowp, NEG, s)
            mnew = jnp.maximum(m0, jnp.max(s, axis=-1, keepdims=True))
            alpha = jnp.exp(m0 - mnew)
            pexp = jnp.exp(s - mnew)
            vblk = v_s[pl.ds(kt * TQ, TQ), :]
            acc = acc * alpha + jax.lax.dot_general(
                pexp, vblk, (((1,), (0,)), ((), ())),
                preferred_element_type=jnp.float32)
            l0 = l0 * alpha + jnp.sum(pexp, axis=-1, keepdims=True)
            return acc, mnew, l0

        acc0 = jnp.zeros((TQ, DH), jnp.float32)
        m0 = jnp.full((TQ, 1), NEG, jnp.float32)
        l0 = jnp.zeros((TQ, 1), jnp.float32)
        acc, _, l = jax.lax.fori_loop(0, qt + 1, body, (acc0, m0, l0))
        local = acc / l

        vtile = v_s[pl.ds(qt * TQ, TQ), :]
        rtile = read[qt * TQ:(qt + 1) * TQ, :]
        gl = jax.lax.dot_general(qtile, wf_ref[...], (((1,), (1,)), ((), ())),
                                 preferred_element_type=jnp.float32) + bf_ref[...]
        gmx = jnp.max(gl, axis=-1, keepdims=True)
        ge = jnp.exp(gl - gmx)
        g = ge / jnp.sum(ge, axis=-1, keepdims=True)
        yh_s[pl.ds(qt * TQ, TQ), :] = (g[:, 0:1] * rtile + g[:, 1:2] * local
                                       + g[:, 2:3] * vtile)

    contrib = jax.lax.dot_general(yh_s[...], wo_ref[...],
                                  (((1,), (0,)), ((), ())),
                                  preferred_element_type=jnp.float32)

    @pl.when(h == 0)
    def _init():
        o_ref[0] = contrib

    @pl.when(h != 0)
    def _acc():
        o_ref[0] = o_ref[0] + contrib


@jax.jit
def kernel(x, Wqkv, Wout, slot_k_init, slot_v_init, Wg, bg, Wf, bf,
           log_tau_read, log_tau_write):
    ltau = log_tau_read.reshape(1, 1)
    bf2 = bf.reshape(1, 3)
    WoT = Wout.T

    y = pl.pallas_call(
        _attn_kernel,
        grid=(B, H),
        in_specs=[
            pl.BlockSpec((1, 1), lambda b, h: (0, 0)),
            pl.BlockSpec((1, T, D), lambda b, h: (b, 0, 0)),
            pl.BlockSpec((DH, D), lambda b, h: (h, 0)),
            pl.BlockSpec((DH, D), lambda b, h: (H + h, 0)),
            pl.BlockSpec((DH, D), lambda b, h: (2 * H + h, 0)),
            pl.BlockSpec((1, K, DH), lambda b, h: (h, 0, 0)),
            pl.BlockSpec((1, K, DH), lambda b, h: (h, 0, 0)),
            pl.BlockSpec((3, DH), lambda b, h: (0, 0)),
            pl.BlockSpec((1, 3), lambda b, h: (0, 0)),
            pl.BlockSpec((DH, D), lambda b, h: (h, 0)),
        ],
        out_specs=pl.BlockSpec((1, T, D), lambda b, h: (b, 0, 0)),
        out_shape=jax.ShapeDtypeStruct((B, T, D), jnp.float32),
        scratch_shapes=[
            pltpu.VMEM((T, DH), jnp.float32),
            pltpu.VMEM((T, DH), jnp.float32),
            pltpu.VMEM((T, DH), jnp.float32),
            pltpu.VMEM((T, DH), jnp.float32),
        ],
    )(ltau, x, Wqkv, Wqkv, Wqkv, slot_k_init, slot_v_init, Wf, bf2, WoT)
    return y
